# baseline (speedup 1.0000x reference)
import functools

import jax
import jax.numpy as jnp
from jax import lax
from jax.experimental import pallas as pl
from jax.experimental.pallas import tpu as pltpu
from jax.experimental.pallas import tpu_sc as plsc

_NUM_USERS = 500000
_EMB = 64
_B = 16384
_NC = 2
_NS = 16
_NW = _NC * _NS
_BPW = _B // _NW
_L = 16

_NCHUNK = 8
_CS = _BPW // _NCHUNK


def _body(users_hbm, items_hbm, neg_hbm, table_hbm,
          u_out, v_out, n_out,
          idx_u, idx_i, idx_n, rows_u, rows_i, rows_n,
          *sems):
    csems, sem_o = sems[:_NCHUNK], sems[_NCHUNK]
    wid = lax.axis_index("s") * _NC + lax.axis_index("c")
    base = wid * _BPW

    pltpu.sync_copy(users_hbm.at[pl.ds(base, _BPW)], idx_u)
    pltpu.sync_copy(items_hbm.at[pl.ds(base, _BPW)], idx_i)
    pltpu.sync_copy(neg_hbm.at[pl.ds(base, _BPW)], idx_n)
    for j in range(_BPW // _L):
        s = pl.ds(j * _L, _L)
        idx_i[s] = idx_i[s] + _NUM_USERS
        idx_n[s] = idx_n[s] + _NUM_USERS

    cps = []
    for c in range(_NCHUNK):
        s = pl.ds(c * _CS, _CS)
        cps.append((
            pltpu.async_copy(table_hbm.at[idx_u.at[s]], rows_u.at[s], csems[c]),
            pltpu.async_copy(table_hbm.at[idx_i.at[s]], rows_i.at[s], csems[c]),
            pltpu.async_copy(table_hbm.at[idx_n.at[s]], rows_n.at[s], csems[c]),
        ))
    ocps = []
    for c in range(_NCHUNK):
        s = pl.ds(c * _CS, _CS)
        so = pl.ds(base + c * _CS, _CS)
        for cp in cps[c]:
            cp.wait()
        ocps.append(pltpu.async_copy(rows_u.at[s], u_out.at[so], sem_o))
        ocps.append(pltpu.async_copy(rows_i.at[s], v_out.at[so], sem_o))
        ocps.append(pltpu.async_copy(rows_n.at[s], n_out.at[so], sem_o))
    for ocp in ocps:
        ocp.wait()


_gather = functools.partial(
    pl.kernel,
    mesh=plsc.VectorSubcoreMesh(core_axis_name="c", subcore_axis_name="s"),
    compiler_params=pltpu.CompilerParams(use_tc_tiling_on_sc=False),
    out_type=[jax.ShapeDtypeStruct((_B, _EMB), jnp.float32)] * 3,
    scratch_types=[
        pltpu.VMEM((_BPW,), jnp.int32),
        pltpu.VMEM((_BPW,), jnp.int32),
        pltpu.VMEM((_BPW,), jnp.int32),
        pltpu.VMEM((_BPW, _EMB), jnp.float32),
        pltpu.VMEM((_BPW, _EMB), jnp.float32),
        pltpu.VMEM((_BPW, _EMB), jnp.float32),
    ] + [pltpu.SemaphoreType.DMA] * (_NCHUNK + 1),
)(_body)


def kernel(users, items, neg_items, U_and_V):
    u, v, n = _gather(users.astype(jnp.int32), items.astype(jnp.int32),
                      neg_items.astype(jnp.int32), U_and_V)
    return (u, v, n)
